# baseline (speedup 1.0000x reference)
import functools

import jax
import jax.numpy as jnp
from jax import lax
from jax.experimental import pallas as pl
from jax.experimental.pallas import tpu as pltpu
from jax.experimental.pallas import tpu_sc as plsc

R = 128
N = 32768
L = 16
NBITS = 8
D = 1 << NBITS
NPASS = 32 // NBITS
NC, NS = 2, 16
PG = 8
P_TOP = 0.9


def _digit(t, shift):
    return lax.shift_right_logical(t, jnp.int32(shift)) & jnp.int32(D - 1)


def _make_body(rows, n):
    nv = n // L
    chunk = n // L
    chunk_bits = chunk.bit_length() - 1
    rows_per_w = rows // (NC * NS)

    def _sc_body(x_hbm, vals_hbm, order_hbm, bufA, bufB, keysT, counters,
                 psums):
        lane = lax.broadcasted_iota(jnp.int32, (L,), 0)
        ones = jnp.ones((L,), jnp.int32)
        wid = lax.axis_index("s") * NC + lax.axis_index("c")

        def kslot(idx):
            return idx + lax.shift_right_logical(idx, jnp.int32(chunk_bits))

        def row_body(j, _):
            r = wid * rows_per_w + j
            pltpu.sync_copy(x_hbm.at[r], bufA)

            def tr_body(i, mx):
                v = bufA[pl.ds(i * L, L)]
                u = plsc.bitcast(v, jnp.int32)
                t = jnp.where(u < 0, u, ~u & jnp.int32(0x7FFFFFFF))
                e = i * L + lane
                plsc.store_scatter(keysT, [kslot(e)], t)
                return jnp.maximum(mx, v)

            mx16 = lax.fori_loop(0, nv, tr_body,
                                 jnp.full((L,), -jnp.inf, jnp.float32),
                                 unroll=4)
            mx = jnp.max(mx16)

            def sum_body(i, acc):
                v = bufA[pl.ds(i * L, L)]
                return acc + jnp.exp(v - mx)

            acc16 = lax.fori_loop(0, nv, sum_body, jnp.zeros((L,), jnp.float32),
                                  unroll=4)
            total = jnp.sum(acc16)

            for p in range(NPASS):
                shift = p * NBITS
                last = p == NPASS - 1

                def zero_body(i, _):
                    counters[pl.ds(i * L, L)] = jnp.zeros((L,), jnp.int32)
                    return 0

                lax.fori_loop(0, D, zero_body, 0)

                if p == 0:
                    def hist_body(i, _, shift=shift):
                        t = plsc.load_gather(keysT, [lane * (chunk + 1) + i])
                        d = _digit(t, shift)
                        plsc.addupdate_scatter(counters, [d * L + lane], ones)
                        return 0
                else:
                    src = bufA if p % 2 == 1 else bufB

                    def hist_body(i, _, src=src, shift=shift):
                        iv = src[pl.ds(i * L, L)]
                        idxv = (plsc.bitcast(iv, jnp.int32)
                                if src is bufA else iv)
                        t = plsc.load_gather(keysT, [kslot(idxv)])
                        d = _digit(t, shift)
                        plsc.addupdate_scatter(counters, [d * L + lane], ones)
                        return 0

                lax.fori_loop(0, nv, hist_body, 0, unroll=4)

                def scan_body(i, carry):
                    v = counters[pl.ds(i * L, L)]
                    inc = plsc.cumsum(v)
                    counters[pl.ds(i * L, L)] = inc - v + carry
                    return carry + jnp.max(inc)

                lax.fori_loop(0, D, scan_body, jnp.int32(0))

                def perm_group(g, _, p=p, shift=shift, last=last):
                    ds_, idxs = [], []
                    for k in range(PG):
                        i = g * PG + k
                        if p == 0:
                            t = plsc.load_gather(
                                keysT, [lane * (chunk + 1) + i])
                            idxv = lane * chunk + i
                        else:
                            src = bufA if p % 2 == 1 else bufB
                            iv = src[pl.ds(i * L, L)]
                            idxv = (plsc.bitcast(iv, jnp.int32)
                                    if src is bufA else iv)
                            t = plsc.load_gather(keysT, [kslot(idxv)])
                        ds_.append(_digit(t, shift))
                        idxs.append(idxv)
                    bases = [plsc.load_gather(counters, [d * L + lane])
                             for d in ds_]
                    zero = jnp.zeros((L,), jnp.int32)
                    for k in range(PG):
                        occ = zero
                        for j in range(k):
                            occ = occ + jnp.where(ds_[j] == ds_[k], 1, 0)
                        pos = bases[k] + occ
                        is_last = jnp.full((L,), True)
                        for j in range(k + 1, PG):
                            is_last = is_last & (ds_[j] != ds_[k])
                        plsc.store_scatter(counters, [ds_[k] * L + lane],
                                           pos + ones, mask=is_last)
                        if last:
                            s = pos
                        else:
                            s = ((pos & jnp.int32(chunk - 1)) << 4) | \
                                lax.shift_right_logical(
                                    pos, jnp.int32(chunk_bits))
                        if p % 2 == 0:
                            plsc.store_scatter(bufA, [s],
                                               plsc.bitcast(idxs[k],
                                                            jnp.float32))
                        else:
                            plsc.store_scatter(bufB, [s], idxs[k])
                    return 0

                lax.fori_loop(0, nv // PG, perm_group, 0)

            thresh = jnp.float32(P_TOP) * total

            def mval_body(i, _):
                idxv = bufB[pl.ds(i * L, L)]
                t = plsc.load_gather(keysT, [kslot(idxv)])
                u = jnp.where(t < 0, t, ~t & jnp.int32(0x7FFFFFFF))
                v = plsc.bitcast(u, jnp.float32)
                bufA[pl.ds(i * L, L)] = v
                ssum = jnp.zeros((L,), jnp.float32) + jnp.sum(jnp.exp(v - mx))
                plsc.store_scatter(psums, [lane * 0 + i], ssum,
                                   mask=lane == 0)
                return 0

            lax.fori_loop(0, nv, mval_body, 0, unroll=4)

            def pfx_body(i, carry):
                v = psums[pl.ds(i * L, L)]
                inc = plsc.cumsum(v)
                psums[pl.ds(i * L, L)] = inc - v + carry
                return carry + jnp.max(inc)

            lax.fori_loop(0, nv // L, pfx_body, jnp.float32(0.0))

            def mask_body(i, _):
                v = bufA[pl.ds(i * L, L)]
                e = jnp.exp(v - mx)
                inc = plsc.cumsum(e)
                base = plsc.load_gather(psums, [lane * 0 + i])
                excl = inc - e + base
                out = jnp.where(excl < thresh, v, -jnp.inf)
                bufA[pl.ds(i * L, L)] = out
                return 0

            lax.fori_loop(0, nv, mask_body, 0, unroll=4)

            pltpu.sync_copy(bufA, vals_hbm.at[r])
            pltpu.sync_copy(bufB, order_hbm.at[r])
            return 0

        lax.fori_loop(0, rows_per_w, row_body, 0)

    return _sc_body


def _make_kernel(rows, n, interpret=False):
    mesh = plsc.VectorSubcoreMesh(core_axis_name="c", subcore_axis_name="s",
                                  num_cores=NC, num_subcores=NS)
    return pl.kernel(
        _make_body(rows, n),
        out_type=(
            jax.ShapeDtypeStruct((rows, n), jnp.float32),
            jax.ShapeDtypeStruct((rows, n), jnp.int32),
        ),
        mesh=mesh,
        scratch_types=[
            pltpu.VMEM((n,), jnp.float32),
            pltpu.VMEM((n,), jnp.int32),
            pltpu.VMEM((n + L,), jnp.int32),
            pltpu.VMEM((D * L,), jnp.int32),
            pltpu.VMEM((n // L,), jnp.float32),
        ],
        compiler_params=pltpu.CompilerParams(needs_layout_passes=False),
        interpret=interpret,
    )


@jax.jit
def kernel(x):
    return _make_kernel(R, N)(x)


# speedup vs baseline: 1.0850x; 1.0850x over previous
import functools

import jax
import jax.numpy as jnp
from jax import lax
from jax.experimental import pallas as pl
from jax.experimental.pallas import tpu as pltpu
from jax.experimental.pallas import tpu_sc as plsc

R = 128
N = 32768
L = 16
NBITS = 8
D = 1 << NBITS
NPASS = 32 // NBITS
NC, NS = 2, 16
PG = 8
P_TOP = 0.9


def _digit(t, shift):
    return lax.shift_right_logical(t, jnp.int32(shift)) & jnp.int32(D - 1)


def _make_body(rows, n):
    nv = n // L
    chunk = n // L
    chunk_bits = chunk.bit_length() - 1
    rows_per_w = rows // (NC * NS)

    def _sc_body(x_hbm, vals_hbm, order_hbm, bufA, bufB, keysT, counters,
                 psums):
        lane = lax.broadcasted_iota(jnp.int32, (L,), 0)
        ones = jnp.ones((L,), jnp.int32)
        wid = lax.axis_index("s") * NC + lax.axis_index("c")

        def kslot(idx):
            return idx + lax.shift_right_logical(idx, jnp.int32(chunk_bits))

        def row_body(j, _):
            r = wid * rows_per_w + j
            pltpu.sync_copy(x_hbm.at[r], bufA)

            def tr_body(i, mx):
                v = bufA[pl.ds(i * L, L)]
                u = plsc.bitcast(v, jnp.int32)
                t = jnp.where(u < 0, u, ~u & jnp.int32(0x7FFFFFFF))
                e = i * L + lane
                plsc.store_scatter(keysT, [kslot(e)], t)
                return jnp.maximum(mx, v)

            mx16 = lax.fori_loop(0, nv, tr_body,
                                 jnp.full((L,), -jnp.inf, jnp.float32),
                                 unroll=4)
            mx = jnp.max(mx16)

            def sum_body(i, acc):
                v = bufA[pl.ds(i * L, L)]
                return acc + jnp.exp(v - mx)

            acc16 = lax.fori_loop(0, nv, sum_body, jnp.zeros((L,), jnp.float32),
                                  unroll=4)
            total = jnp.sum(acc16)

            for p in range(NPASS):
                shift = p * NBITS
                last = p == NPASS - 1

                def zero_body(i, _):
                    counters[pl.ds(i * L, L)] = jnp.zeros((L,), jnp.int32)
                    return 0

                lax.fori_loop(0, D, zero_body, 0)

                if p == 0:
                    def hist_body(i, _, shift=shift):
                        t = plsc.load_gather(keysT, [lane * (chunk + 1) + i])
                        d = _digit(t, shift)
                        plsc.addupdate_scatter(counters, [d * L + lane], ones)
                        return 0
                else:
                    src = bufA if p % 2 == 1 else bufB

                    def hist_body(i, _, src=src, shift=shift):
                        iv = src[pl.ds(i * L, L)]
                        idxv = (plsc.bitcast(iv, jnp.int32)
                                if src is bufA else iv)
                        t = plsc.load_gather(keysT, [kslot(idxv)])
                        d = _digit(t, shift)
                        plsc.addupdate_scatter(counters, [d * L + lane], ones)
                        return 0

                lax.fori_loop(0, nv, hist_body, 0, unroll=4)

                def scan_body(i, carry):
                    v = counters[pl.ds(i * L, L)]
                    inc = plsc.cumsum(v)
                    counters[pl.ds(i * L, L)] = inc - v + carry
                    return carry + jnp.max(inc)

                lax.fori_loop(0, D, scan_body, jnp.int32(0))

                def perm_group(g, _, p=p, shift=shift, last=last):
                    ds_, idxs = [], []
                    for k in range(PG):
                        i = g * PG + k
                        if p == 0:
                            t = plsc.load_gather(
                                keysT, [lane * (chunk + 1) + i])
                            idxv = lane * chunk + i
                        else:
                            src = bufA if p % 2 == 1 else bufB
                            iv = src[pl.ds(i * L, L)]
                            idxv = (plsc.bitcast(iv, jnp.int32)
                                    if src is bufA else iv)
                            t = plsc.load_gather(keysT, [kslot(idxv)])
                        ds_.append(_digit(t, shift))
                        idxs.append(idxv)
                    bases = [plsc.load_gather(counters, [d * L + lane])
                             for d in ds_]
                    zero = jnp.zeros((L,), jnp.int32)
                    for k in range(PG):
                        occ = zero
                        for j in range(k):
                            occ = occ + jnp.where(ds_[j] == ds_[k], 1, 0)
                        pos = bases[k] + occ
                        is_last = jnp.full((L,), True)
                        for j in range(k + 1, PG):
                            is_last = is_last & (ds_[j] != ds_[k])
                        plsc.store_scatter(counters, [ds_[k] * L + lane],
                                           pos + ones, mask=is_last)
                        if last:
                            s = pos
                        else:
                            s = ((pos & jnp.int32(chunk - 1)) << 4) | \
                                lax.shift_right_logical(
                                    pos, jnp.int32(chunk_bits))
                        if p % 2 == 0:
                            plsc.store_scatter(bufA, [s],
                                               plsc.bitcast(idxs[k],
                                                            jnp.float32))
                        else:
                            plsc.store_scatter(bufB, [s], idxs[k])
                    return 0

                lax.fori_loop(0, nv // PG, perm_group, 0)

            thresh = jnp.float32(P_TOP) * total

            def mask_body(i, cum):
                idxv = bufB[pl.ds(i * L, L)]
                t = plsc.load_gather(keysT, [kslot(idxv)])
                u = jnp.where(t < 0, t, ~t & jnp.int32(0x7FFFFFFF))
                v = plsc.bitcast(u, jnp.float32)
                e = jnp.exp(v - mx)
                inc = plsc.cumsum(e)
                excl = inc - e + cum
                out = jnp.where(excl < thresh, v, -jnp.inf)
                bufA[pl.ds(i * L, L)] = out
                return cum + jnp.max(inc)

            lax.fori_loop(0, nv, mask_body, jnp.float32(0.0))

            pltpu.sync_copy(bufA, vals_hbm.at[r])
            pltpu.sync_copy(bufB, order_hbm.at[r])
            return 0

        lax.fori_loop(0, rows_per_w, row_body, 0)

    return _sc_body


def _make_kernel(rows, n, interpret=False):
    mesh = plsc.VectorSubcoreMesh(core_axis_name="c", subcore_axis_name="s",
                                  num_cores=NC, num_subcores=NS)
    return pl.kernel(
        _make_body(rows, n),
        out_type=(
            jax.ShapeDtypeStruct((rows, n), jnp.float32),
            jax.ShapeDtypeStruct((rows, n), jnp.int32),
        ),
        mesh=mesh,
        scratch_types=[
            pltpu.VMEM((n,), jnp.float32),
            pltpu.VMEM((n,), jnp.int32),
            pltpu.VMEM((n + L,), jnp.int32),
            pltpu.VMEM((D * L,), jnp.int32),
            pltpu.VMEM((n // L,), jnp.float32),
        ],
        compiler_params=pltpu.CompilerParams(needs_layout_passes=False),
        interpret=interpret,
    )


@jax.jit
def kernel(x):
    return _make_kernel(R, N)(x)


# speedup vs baseline: 1.2516x; 1.1535x over previous
import functools

import jax
import jax.numpy as jnp
from jax import lax
from jax.experimental import pallas as pl
from jax.experimental.pallas import tpu as pltpu
from jax.experimental.pallas import tpu_sc as plsc

R = 128
N = 32768
L = 16
NBITS = 8
D = 1 << NBITS
NPASS = 32 // NBITS
NC, NS = 2, 16
PG = 4
P_TOP = 0.9


def _digit(t, shift):
    return lax.shift_right_logical(t, jnp.int32(shift)) & jnp.int32(D - 1)


def _make_body(rows, n):
    nv = n // L
    chunk = n // L
    chunk_bits = chunk.bit_length() - 1
    rows_per_w = rows // (NC * NS)

    def _sc_body(x_hbm, vals_hbm, order_hbm, bufA, bufB, keysT, counters,
                 psums):
        lane = lax.broadcasted_iota(jnp.int32, (L,), 0)
        ones = jnp.ones((L,), jnp.int32)
        wid = lax.axis_index("s") * NC + lax.axis_index("c")

        def kslot(idx):
            return idx + lax.shift_right_logical(idx, jnp.int32(chunk_bits))

        def row_body(j, _):
            r = wid * rows_per_w + j
            pltpu.sync_copy(x_hbm.at[r], bufA)

            def tr_body(i, mx):
                v = bufA[pl.ds(i * L, L)]
                u = plsc.bitcast(v, jnp.int32)
                t = jnp.where(u < 0, u, ~u & jnp.int32(0x7FFFFFFF))
                e = i * L + lane
                plsc.store_scatter(keysT, [kslot(e)], t)
                return jnp.maximum(mx, v)

            mx16 = lax.fori_loop(0, nv, tr_body,
                                 jnp.full((L,), -jnp.inf, jnp.float32),
                                 unroll=4)
            mx = jnp.max(mx16)

            def sum_body(i, acc):
                v = bufA[pl.ds(i * L, L)]
                return acc + jnp.exp(v - mx)

            acc16 = lax.fori_loop(0, nv, sum_body, jnp.zeros((L,), jnp.float32),
                                  unroll=4)
            total = jnp.sum(acc16)

            for p in range(NPASS):
                shift = p * NBITS
                last = p == NPASS - 1

                def zero_body(i, _):
                    counters[pl.ds(i * L, L)] = jnp.zeros((L,), jnp.int32)
                    return 0

                lax.fori_loop(0, D, zero_body, 0)

                if p == 0:
                    def hist_body(i, _, shift=shift):
                        t = plsc.load_gather(keysT, [lane * (chunk + 1) + i])
                        d = _digit(t, shift)
                        plsc.addupdate_scatter(counters, [d * L + lane], ones)
                        return 0
                else:
                    src = bufA if p % 2 == 1 else bufB

                    def hist_body(i, _, src=src, shift=shift):
                        iv = src[pl.ds(i * L, L)]
                        idxv = (plsc.bitcast(iv, jnp.int32)
                                if src is bufA else iv)
                        t = plsc.load_gather(keysT, [kslot(idxv)])
                        d = _digit(t, shift)
                        plsc.addupdate_scatter(counters, [d * L + lane], ones)
                        return 0

                lax.fori_loop(0, nv, hist_body, 0, unroll=4)

                def scan_body(i, carry):
                    v = counters[pl.ds(i * L, L)]
                    inc = plsc.cumsum(v)
                    counters[pl.ds(i * L, L)] = inc - v + carry
                    return carry + jnp.max(inc)

                lax.fori_loop(0, D, scan_body, jnp.int32(0))

                def perm_group(g, _, p=p, shift=shift, last=last):
                    ds_, idxs = [], []
                    for k in range(PG):
                        i = g * PG + k
                        if p == 0:
                            t = plsc.load_gather(
                                keysT, [lane * (chunk + 1) + i])
                            idxv = lane * chunk + i
                        else:
                            src = bufA if p % 2 == 1 else bufB
                            iv = src[pl.ds(i * L, L)]
                            idxv = (plsc.bitcast(iv, jnp.int32)
                                    if src is bufA else iv)
                            t = plsc.load_gather(keysT, [kslot(idxv)])
                        ds_.append(_digit(t, shift))
                        idxs.append(idxv)
                    bases = [plsc.load_gather(counters, [d * L + lane])
                             for d in ds_]
                    zero = jnp.zeros((L,), jnp.int32)
                    for k in range(PG):
                        occ = zero
                        for j in range(k):
                            occ = occ + jnp.where(ds_[j] == ds_[k], 1, 0)
                        pos = bases[k] + occ
                        is_last = jnp.full((L,), True)
                        for j in range(k + 1, PG):
                            is_last = is_last & (ds_[j] != ds_[k])
                        plsc.store_scatter(counters, [ds_[k] * L + lane],
                                           pos + ones, mask=is_last)
                        if last:
                            s = pos
                        else:
                            s = ((pos & jnp.int32(chunk - 1)) << 4) | \
                                lax.shift_right_logical(
                                    pos, jnp.int32(chunk_bits))
                        if p % 2 == 0:
                            plsc.store_scatter(bufA, [s],
                                               plsc.bitcast(idxs[k],
                                                            jnp.float32))
                        else:
                            plsc.store_scatter(bufB, [s], idxs[k])
                    return 0

                lax.fori_loop(0, nv // PG, perm_group, 0)

            thresh = jnp.float32(P_TOP) * total

            MG = 4

            def mask_body(g, cum):
                vs, es, incs = [], [], []
                for k in range(MG):
                    i = g * MG + k
                    idxv = bufB[pl.ds(i * L, L)]
                    t = plsc.load_gather(keysT, [kslot(idxv)])
                    u = jnp.where(t < 0, t, ~t & jnp.int32(0x7FFFFFFF))
                    v = plsc.bitcast(u, jnp.float32)
                    e = jnp.exp(v - mx)
                    vs.append(v)
                    es.append(e)
                    incs.append(plsc.cumsum(e))
                for k in range(MG):
                    i = g * MG + k
                    excl = incs[k] - es[k] + cum
                    out = jnp.where(excl < thresh, vs[k], -jnp.inf)
                    bufA[pl.ds(i * L, L)] = out
                    cum = cum + jnp.max(incs[k])
                return cum

            lax.fori_loop(0, nv // MG, mask_body, jnp.float32(0.0))

            pltpu.sync_copy(bufA, vals_hbm.at[r])
            pltpu.sync_copy(bufB, order_hbm.at[r])
            return 0

        lax.fori_loop(0, rows_per_w, row_body, 0)

    return _sc_body


def _make_kernel(rows, n, interpret=False):
    mesh = plsc.VectorSubcoreMesh(core_axis_name="c", subcore_axis_name="s",
                                  num_cores=NC, num_subcores=NS)
    return pl.kernel(
        _make_body(rows, n),
        out_type=(
            jax.ShapeDtypeStruct((rows, n), jnp.float32),
            jax.ShapeDtypeStruct((rows, n), jnp.int32),
        ),
        mesh=mesh,
        scratch_types=[
            pltpu.VMEM((n,), jnp.float32),
            pltpu.VMEM((n,), jnp.int32),
            pltpu.VMEM((n + L,), jnp.int32),
            pltpu.VMEM((D * L,), jnp.int32),
            pltpu.VMEM((n // L,), jnp.float32),
        ],
        compiler_params=pltpu.CompilerParams(needs_layout_passes=False),
        interpret=interpret,
    )


@jax.jit
def kernel(x):
    return _make_kernel(R, N)(x)


# speedup vs baseline: 1.6568x; 1.3238x over previous
import functools

import jax
import jax.numpy as jnp
from jax import lax
from jax.experimental import pallas as pl
from jax.experimental.pallas import tpu as pltpu
from jax.experimental.pallas import tpu_sc as plsc

R = 128
N = 32768
L = 16
NBITS = 8
D = 1 << NBITS
NPASS = 32 // NBITS
NC, NS = 2, 16
PG = 4
P_TOP = 0.9


def _digit(t, shift):
    return lax.shift_right_logical(t, jnp.int32(shift)) & jnp.int32(D - 1)


def _make_body(rows, n):
    nv = n // L
    chunk = n // L
    chunk_bits = chunk.bit_length() - 1
    rows_per_w = rows // (NC * NS)

    def _sc_body(x_hbm, vals_hbm, order_hbm, bufA, bufB, keysT, counters,
                 psums):
        lane = lax.broadcasted_iota(jnp.int32, (L,), 0)
        ones = jnp.ones((L,), jnp.int32)
        wid = lax.axis_index("s") * NC + lax.axis_index("c")

        def kslot(idx):
            return idx + lax.shift_right_logical(idx, jnp.int32(chunk_bits))

        def row_body(j, _):
            r = wid * rows_per_w + j
            pltpu.sync_copy(x_hbm.at[r], bufA)

            def tr_body(i, mx):
                v = bufA[pl.ds(i * L, L)]
                u = plsc.bitcast(v, jnp.int32)
                t = jnp.where(u < 0, u, ~u & jnp.int32(0x7FFFFFFF))
                e = i * L + lane
                plsc.store_scatter(keysT, [kslot(e)], t)
                return jnp.maximum(mx, v)

            mx16 = lax.fori_loop(0, nv, tr_body,
                                 jnp.full((L,), -jnp.inf, jnp.float32),
                                 unroll=4)
            mx = jnp.max(mx16)

            def sum_body(i, acc):
                v = bufA[pl.ds(i * L, L)]
                return acc + jnp.exp(v - mx)

            acc16 = lax.fori_loop(0, nv, sum_body, jnp.zeros((L,), jnp.float32),
                                  unroll=4)
            total = jnp.sum(acc16)

            c15 = jnp.int32(15)
            c23 = jnp.int32(23)
            cD1 = jnp.int32(D - 1)
            cIM = jnp.int32(n - 1)

            for p in range(NPASS):
                last = p == NPASS - 1
                src = None if p == 0 else (bufA if p % 2 == 1 else bufB)

                def load_pk(i, src=src):
                    iv = src[pl.ds(i * L, L)]
                    return plsc.bitcast(iv, jnp.int32) if src is bufA else iv

                def zero_body(i, _):
                    counters[pl.ds(i * L, L)] = jnp.zeros((L,), jnp.int32)
                    return 0

                lax.fori_loop(0, D, zero_body, 0)

                if p == 0:
                    def hist_body(i, _):
                        t = plsc.load_gather(keysT, [lane * (chunk + 1) + i])
                        d = t & cD1
                        plsc.addupdate_scatter(counters, [d * L + lane], ones)
                        return 0
                else:
                    def hist_body(i, _, load_pk=load_pk):
                        pk = load_pk(i)
                        d = lax.shift_right_logical(pk, c15) & cD1
                        plsc.addupdate_scatter(counters, [d * L + lane], ones)
                        return 0

                lax.fori_loop(0, nv, hist_body, 0, unroll=4 if p == 0 else 8)

                def scan_body(i, carry):
                    v = counters[pl.ds(i * L, L)]
                    inc = plsc.cumsum(v)
                    counters[pl.ds(i * L, L)] = inc - v + carry
                    return carry + jnp.max(inc)

                lax.fori_loop(0, D, scan_body, jnp.int32(0))

                def perm_group(g, _, p=p, last=last, load_pk=load_pk):
                    ds_, outs = [], []
                    for k in range(PG):
                        i = g * PG + k
                        if p == 0:
                            t = plsc.load_gather(
                                keysT, [lane * (chunk + 1) + i])
                            idxv = lane * chunk + i
                            d = t & cD1
                            d1 = lax.shift_right_logical(t, jnp.int32(8)) & cD1
                            d2 = lax.shift_right_logical(t, jnp.int32(16)) & cD1
                            out = (d2 << c23) | (d1 << c15) | idxv
                        else:
                            pk = load_pk(i)
                            d = lax.shift_right_logical(pk, c15) & cD1
                            idxv = pk & cIM
                            if p == 1:
                                sp = lax.shift_right_logical(pk, c23)
                                t = plsc.load_gather(keysT, [kslot(idxv)])
                                d3 = lax.shift_right_logical(t, jnp.int32(24))
                                out = (d3 << c23) | (sp << c15) | idxv
                            elif p == 2:
                                sp = lax.shift_right_logical(pk, c23)
                                out = (sp << c15) | idxv
                            else:
                                out = idxv
                        ds_.append(d)
                        outs.append(out)
                    bases = [plsc.load_gather(counters, [d * L + lane])
                             for d in ds_]
                    for k in range(PG):
                        occ = jnp.zeros((L,), jnp.int32)
                        for j in range(k):
                            occ = occ + jnp.where(ds_[j] == ds_[k], 1, 0)
                        pos = bases[k] + occ
                        is_last = jnp.full((L,), True)
                        for j in range(k + 1, PG):
                            is_last = is_last & (ds_[j] != ds_[k])
                        plsc.store_scatter(counters, [ds_[k] * L + lane],
                                           pos + ones, mask=is_last)
                        if last:
                            s = pos
                        else:
                            s = ((pos & jnp.int32(chunk - 1)) << 4) | \
                                lax.shift_right_logical(
                                    pos, jnp.int32(chunk_bits))
                        if p % 2 == 0:
                            plsc.store_scatter(bufA, [s],
                                               plsc.bitcast(outs[k],
                                                            jnp.float32))
                        else:
                            plsc.store_scatter(bufB, [s], outs[k])
                    return 0

                lax.fori_loop(0, nv // PG, perm_group, 0)

            thresh = jnp.float32(P_TOP) * total

            MG = 4

            def mask_body(g, cum):
                vs, es, incs = [], [], []
                for k in range(MG):
                    i = g * MG + k
                    idxv = bufB[pl.ds(i * L, L)]
                    t = plsc.load_gather(keysT, [kslot(idxv)])
                    u = jnp.where(t < 0, t, ~t & jnp.int32(0x7FFFFFFF))
                    v = plsc.bitcast(u, jnp.float32)
                    e = jnp.exp(v - mx)
                    vs.append(v)
                    es.append(e)
                    incs.append(plsc.cumsum(e))
                for k in range(MG):
                    i = g * MG + k
                    excl = incs[k] - es[k] + cum
                    out = jnp.where(excl < thresh, vs[k], -jnp.inf)
                    bufA[pl.ds(i * L, L)] = out
                    cum = cum + jnp.max(incs[k])
                return cum

            lax.fori_loop(0, nv // MG, mask_body, jnp.float32(0.0))

            pltpu.sync_copy(bufA, vals_hbm.at[r])
            pltpu.sync_copy(bufB, order_hbm.at[r])
            return 0

        lax.fori_loop(0, rows_per_w, row_body, 0)

    return _sc_body


def _make_kernel(rows, n, interpret=False):
    mesh = plsc.VectorSubcoreMesh(core_axis_name="c", subcore_axis_name="s",
                                  num_cores=NC, num_subcores=NS)
    return pl.kernel(
        _make_body(rows, n),
        out_type=(
            jax.ShapeDtypeStruct((rows, n), jnp.float32),
            jax.ShapeDtypeStruct((rows, n), jnp.int32),
        ),
        mesh=mesh,
        scratch_types=[
            pltpu.VMEM((n,), jnp.float32),
            pltpu.VMEM((n,), jnp.int32),
            pltpu.VMEM((n + L,), jnp.int32),
            pltpu.VMEM((D * L,), jnp.int32),
            pltpu.VMEM((n // L,), jnp.float32),
        ],
        compiler_params=pltpu.CompilerParams(needs_layout_passes=False),
        interpret=interpret,
    )


@jax.jit
def kernel(x):
    return _make_kernel(R, N)(x)


# speedup vs baseline: 1.8085x; 1.0915x over previous
import functools

import jax
import jax.numpy as jnp
from jax import lax
from jax.experimental import pallas as pl
from jax.experimental.pallas import tpu as pltpu
from jax.experimental.pallas import tpu_sc as plsc

R = 128
N = 32768
L = 16
NBITS = 8
D = 1 << NBITS
NPASS = 32 // NBITS
NC, NS = 2, 16
PG = 4
P_TOP = 0.9


def _digit(t, shift):
    return lax.shift_right_logical(t, jnp.int32(shift)) & jnp.int32(D - 1)


def _make_body(rows, n):
    nv = n // L
    chunk = n // L
    chunk_bits = chunk.bit_length() - 1
    rows_per_w = rows // (NC * NS)

    def _sc_body(x_hbm, vals_hbm, order_hbm, bufA, bufB, keysT, counters,
                 psums):
        lane = lax.broadcasted_iota(jnp.int32, (L,), 0)
        ones = jnp.ones((L,), jnp.int32)
        wid = lax.axis_index("s") * NC + lax.axis_index("c")

        def kslot(idx):
            return idx + lax.shift_right_logical(idx, jnp.int32(chunk_bits))

        def row_body(j, _):
            r = wid * rows_per_w + j
            pltpu.sync_copy(x_hbm.at[r], bufA)

            def tr_body(i, mx):
                v = bufA[pl.ds(i * L, L)]
                u = plsc.bitcast(v, jnp.int32)
                t = jnp.where(u < 0, u, ~u & jnp.int32(0x7FFFFFFF))
                e = i * L + lane
                plsc.store_scatter(keysT, [kslot(e)], t)
                return jnp.maximum(mx, v)

            mx16 = lax.fori_loop(0, nv, tr_body,
                                 jnp.full((L,), -jnp.inf, jnp.float32),
                                 unroll=4)
            mx = jnp.max(mx16)

            def sum_body(i, acc):
                v = bufA[pl.ds(i * L, L)]
                return acc + jnp.exp(v - mx)

            acc16 = lax.fori_loop(0, nv, sum_body, jnp.zeros((L,), jnp.float32),
                                  unroll=4)
            total = jnp.sum(acc16)

            c15 = jnp.int32(15)
            c23 = jnp.int32(23)
            cD1 = jnp.int32(D - 1)
            cIM = jnp.int32(n - 1)

            for p in range(NPASS):
                last = p == NPASS - 1
                src = None if p == 0 else (bufA if p % 2 == 1 else bufB)

                def load_pk(i, src=src):
                    iv = src[pl.ds(i * L, L)]
                    return plsc.bitcast(iv, jnp.int32) if src is bufA else iv

                def zero_body(i, _):
                    counters[pl.ds(i * L, L)] = jnp.zeros((L,), jnp.int32)
                    return 0

                lax.fori_loop(0, D, zero_body, 0)

                if p == 0:
                    def hist_body(i, _):
                        t = plsc.load_gather(keysT, [lane * (chunk + 1) + i])
                        d = t & cD1
                        plsc.addupdate_scatter(counters, [d * L + lane], ones)
                        return 0
                else:
                    def hist_body(i, _, load_pk=load_pk):
                        pk = load_pk(i)
                        d = lax.shift_right_logical(pk, c15) & cD1
                        plsc.addupdate_scatter(counters, [d * L + lane], ones)
                        return 0

                lax.fori_loop(0, nv, hist_body, 0, unroll=4 if p == 0 else 16)

                def scan_body(g, carry):
                    vs = [counters[pl.ds((g * 4 + k) * L, L)]
                          for k in range(4)]
                    incs = [plsc.cumsum(v) for v in vs]
                    tots = [jnp.max(inc) for inc in incs]
                    for k in range(4):
                        counters[pl.ds((g * 4 + k) * L, L)] = \
                            incs[k] - vs[k] + carry
                        carry = carry + tots[k]
                    return carry

                lax.fori_loop(0, D // 4, scan_body, jnp.int32(0))

                def perm_group(g, _, p=p, last=last, load_pk=load_pk):
                    ds_, outs = [], []
                    for k in range(PG):
                        i = g * PG + k
                        if p == 0:
                            t = plsc.load_gather(
                                keysT, [lane * (chunk + 1) + i])
                            idxv = lane * chunk + i
                            d = t & cD1
                            d1 = lax.shift_right_logical(t, jnp.int32(8)) & cD1
                            d2 = lax.shift_right_logical(t, jnp.int32(16)) & cD1
                            out = (d2 << c23) | (d1 << c15) | idxv
                        else:
                            pk = load_pk(i)
                            d = lax.shift_right_logical(pk, c15) & cD1
                            idxv = pk & cIM
                            if p == 1:
                                sp = lax.shift_right_logical(pk, c23)
                                t = plsc.load_gather(keysT, [kslot(idxv)])
                                d3 = lax.shift_right_logical(t, jnp.int32(24))
                                out = (d3 << c23) | (sp << c15) | idxv
                            elif p == 2:
                                sp = lax.shift_right_logical(pk, c23)
                                out = (sp << c15) | idxv
                            else:
                                out = idxv
                        ds_.append(d)
                        outs.append(out)
                    bases = [plsc.load_gather(counters, [d * L + lane])
                             for d in ds_]
                    eqs = {}
                    for j in range(PG):
                        for k in range(j + 1, PG):
                            eqs[(j, k)] = ds_[j] == ds_[k]
                    for k in range(PG):
                        occ = jnp.zeros((L,), jnp.int32)
                        for j in range(k):
                            occ = occ + jnp.where(eqs[(j, k)], 1, 0)
                        pos = bases[k] + occ
                        is_last = jnp.full((L,), True)
                        for j in range(k + 1, PG):
                            is_last = is_last & jnp.logical_not(eqs[(k, j)])
                        plsc.store_scatter(counters, [ds_[k] * L + lane],
                                           pos + ones, mask=is_last)
                        if last:
                            s = pos
                        else:
                            s = ((pos & jnp.int32(chunk - 1)) << 4) | \
                                lax.shift_right_logical(
                                    pos, jnp.int32(chunk_bits))
                        if p % 2 == 0:
                            plsc.store_scatter(bufA, [s],
                                               plsc.bitcast(outs[k],
                                                            jnp.float32))
                        else:
                            plsc.store_scatter(bufB, [s], outs[k])
                    return 0

                lax.fori_loop(0, nv // PG, perm_group, 0)

            thresh = jnp.float32(P_TOP) * total

            MG = 4
            last_lane = jnp.full((L,), L - 1, jnp.int32)

            def mask_body(g, cum):
                vs, es, incs = [], [], []
                for k in range(MG):
                    i = g * MG + k
                    idxv = bufB[pl.ds(i * L, L)]
                    t = plsc.load_gather(keysT, [kslot(idxv)])
                    u = jnp.where(t < 0, t, ~t & jnp.int32(0x7FFFFFFF))
                    v = plsc.bitcast(u, jnp.float32)
                    e = jnp.exp(v - mx)
                    vs.append(v)
                    es.append(e)
                    incs.append(plsc.cumsum(e))
                for k in range(MG):
                    i = g * MG + k
                    excl = incs[k] - es[k] + cum
                    out = jnp.where(excl < thresh, vs[k], -jnp.inf)
                    bufA[pl.ds(i * L, L)] = out
                    cum = cum + jnp.take(incs[k], last_lane, mode="wrap")
                return cum

            lax.fori_loop(0, nv // MG, mask_body,
                          jnp.zeros((L,), jnp.float32))

            pltpu.sync_copy(bufA, vals_hbm.at[r])
            pltpu.sync_copy(bufB, order_hbm.at[r])
            return 0

        lax.fori_loop(0, rows_per_w, row_body, 0)

    return _sc_body


def _make_kernel(rows, n, interpret=False):
    mesh = plsc.VectorSubcoreMesh(core_axis_name="c", subcore_axis_name="s",
                                  num_cores=NC, num_subcores=NS)
    return pl.kernel(
        _make_body(rows, n),
        out_type=(
            jax.ShapeDtypeStruct((rows, n), jnp.float32),
            jax.ShapeDtypeStruct((rows, n), jnp.int32),
        ),
        mesh=mesh,
        scratch_types=[
            pltpu.VMEM((n,), jnp.float32),
            pltpu.VMEM((n,), jnp.int32),
            pltpu.VMEM((n + L,), jnp.int32),
            pltpu.VMEM((D * L,), jnp.int32),
            pltpu.VMEM((n // L,), jnp.float32),
        ],
        compiler_params=pltpu.CompilerParams(needs_layout_passes=False),
        interpret=interpret,
    )


@jax.jit
def kernel(x):
    return _make_kernel(R, N)(x)


# speedup vs baseline: 1.8767x; 1.0377x over previous
import functools

import jax
import jax.numpy as jnp
from jax import lax
from jax.experimental import pallas as pl
from jax.experimental.pallas import tpu as pltpu
from jax.experimental.pallas import tpu_sc as plsc

R = 128
N = 32768
L = 16
NBITS = 8
D = 1 << NBITS
NPASS = 32 // NBITS
NC, NS = 2, 16
PG = 4
P_TOP = 0.9


def _digit(t, shift):
    return lax.shift_right_logical(t, jnp.int32(shift)) & jnp.int32(D - 1)


def _make_body(rows, n):
    nv = n // L
    chunk = n // L
    chunk_bits = chunk.bit_length() - 1
    rows_per_w = rows // (NC * NS)

    def _sc_body(x_hbm, vals_hbm, order_hbm, bufA, bufB, keysT, counters,
                 psums):
        lane = lax.broadcasted_iota(jnp.int32, (L,), 0)
        ones = jnp.ones((L,), jnp.int32)
        wid = lax.axis_index("s") * NC + lax.axis_index("c")

        def kslot(idx):
            return idx + lax.shift_right_logical(idx, jnp.int32(chunk_bits))

        def row_body(j, _):
            r = wid * rows_per_w + j
            pltpu.sync_copy(x_hbm.at[r], bufA)

            def tr_body(i, mx):
                v = bufA[pl.ds(i * L, L)]
                u = plsc.bitcast(v, jnp.int32)
                t = jnp.where(u < 0, u, ~u & jnp.int32(0x7FFFFFFF))
                e = i * L + lane
                plsc.store_scatter(keysT, [kslot(e)], t)
                return jnp.maximum(mx, v)

            mx16 = lax.fori_loop(0, nv, tr_body,
                                 jnp.full((L,), -jnp.inf, jnp.float32),
                                 unroll=8)
            mx = jnp.max(mx16)

            def sum_body(i, acc):
                v = bufA[pl.ds(i * L, L)]
                return acc + jnp.exp(v - mx)

            acc16 = lax.fori_loop(0, nv, sum_body, jnp.zeros((L,), jnp.float32),
                                  unroll=4)
            total = jnp.sum(acc16)

            c15 = jnp.int32(15)
            c23 = jnp.int32(23)
            cD1 = jnp.int32(D - 1)
            cIM = jnp.int32(n - 1)

            for p in range(NPASS):
                last = p == NPASS - 1
                src = None if p == 0 else (bufA if p % 2 == 1 else bufB)

                def load_pk(i, src=src):
                    iv = src[pl.ds(i * L, L)]
                    return plsc.bitcast(iv, jnp.int32) if src is bufA else iv

                def zero_body(i, _):
                    counters[pl.ds(i * L, L)] = jnp.zeros((L,), jnp.int32)
                    return 0

                lax.fori_loop(0, D, zero_body, 0)

                if p == 0:
                    def hist_body(i, _):
                        t = plsc.load_gather(keysT, [lane * (chunk + 1) + i])
                        d = t & cD1
                        plsc.addupdate_scatter(counters, [d * L + lane], ones)
                        return 0
                else:
                    def hist_body(i, _, load_pk=load_pk):
                        pk = load_pk(i)
                        d = lax.shift_right_logical(pk, c15) & cD1
                        plsc.addupdate_scatter(counters, [d * L + lane], ones)
                        return 0

                lax.fori_loop(0, nv, hist_body, 0, unroll=4 if p == 0 else 16)

                def scan_body(g, carry):
                    vs = [counters[pl.ds((g * 4 + k) * L, L)]
                          for k in range(4)]
                    incs = [plsc.cumsum(v) for v in vs]
                    tots = [jnp.max(inc) for inc in incs]
                    for k in range(4):
                        counters[pl.ds((g * 4 + k) * L, L)] = \
                            incs[k] - vs[k] + carry
                        carry = carry + tots[k]
                    return carry

                lax.fori_loop(0, D // 4, scan_body, jnp.int32(0))

                def perm_group(g, _, p=p, last=last, load_pk=load_pk):
                    ds_, outs = [], []
                    for k in range(PG):
                        i = g * PG + k
                        if p == 0:
                            t = plsc.load_gather(
                                keysT, [lane * (chunk + 1) + i])
                            idxv = lane * chunk + i
                            d = t & cD1
                            d1 = lax.shift_right_logical(t, jnp.int32(8)) & cD1
                            d2 = lax.shift_right_logical(t, jnp.int32(16)) & cD1
                            out = (d2 << c23) | (d1 << c15) | idxv
                        else:
                            pk = load_pk(i)
                            d = lax.shift_right_logical(pk, c15) & cD1
                            idxv = pk & cIM
                            if p == 1:
                                sp = lax.shift_right_logical(pk, c23)
                                t = plsc.load_gather(keysT, [kslot(idxv)])
                                d3 = lax.shift_right_logical(t, jnp.int32(24))
                                out = (d3 << c23) | (sp << c15) | idxv
                            elif p == 2:
                                sp = lax.shift_right_logical(pk, c23)
                                out = (sp << c15) | idxv
                            else:
                                out = idxv
                        ds_.append(d)
                        outs.append(out)
                    bases = [plsc.load_gather(counters, [d * L + lane])
                             for d in ds_]
                    eqs = {}
                    for j in range(PG):
                        for k in range(j + 1, PG):
                            eqs[(j, k)] = ds_[j] == ds_[k]
                    for k in range(PG):
                        occ = jnp.zeros((L,), jnp.int32)
                        for j in range(k):
                            occ = occ + jnp.where(eqs[(j, k)], 1, 0)
                        pos = bases[k] + occ
                        is_last = jnp.full((L,), True)
                        for j in range(k + 1, PG):
                            is_last = is_last & jnp.logical_not(eqs[(k, j)])
                        plsc.store_scatter(counters, [ds_[k] * L + lane],
                                           pos + ones, mask=is_last)
                        if last:
                            s = pos
                        else:
                            s = ((pos & jnp.int32(chunk - 1)) << 4) | \
                                lax.shift_right_logical(
                                    pos, jnp.int32(chunk_bits))
                        if p % 2 == 0:
                            plsc.store_scatter(bufA, [s],
                                               plsc.bitcast(outs[k],
                                                            jnp.float32))
                        else:
                            plsc.store_scatter(bufB, [s], outs[k])
                    return 0

                lax.fori_loop(0, nv // PG, perm_group, 0)

            thresh = jnp.float32(P_TOP) * total

            MG = 8
            last_lane = jnp.full((L,), L - 1, jnp.int32)

            def mask_body(g, cum):
                vs, es, incs = [], [], []
                for k in range(MG):
                    i = g * MG + k
                    idxv = bufB[pl.ds(i * L, L)]
                    t = plsc.load_gather(keysT, [kslot(idxv)])
                    u = jnp.where(t < 0, t, ~t & jnp.int32(0x7FFFFFFF))
                    v = plsc.bitcast(u, jnp.float32)
                    e = jnp.exp(v - mx)
                    vs.append(v)
                    es.append(e)
                    incs.append(plsc.cumsum(e))
                for k in range(MG):
                    i = g * MG + k
                    excl = incs[k] - es[k] + cum
                    out = jnp.where(excl < thresh, vs[k], -jnp.inf)
                    bufA[pl.ds(i * L, L)] = out
                    cum = cum + jnp.take(incs[k], last_lane, mode="wrap")
                return cum

            lax.fori_loop(0, nv // MG, mask_body,
                          jnp.zeros((L,), jnp.float32))

            pltpu.sync_copy(bufA, vals_hbm.at[r])
            pltpu.sync_copy(bufB, order_hbm.at[r])
            return 0

        lax.fori_loop(0, rows_per_w, row_body, 0)

    return _sc_body


def _make_kernel(rows, n, interpret=False):
    mesh = plsc.VectorSubcoreMesh(core_axis_name="c", subcore_axis_name="s",
                                  num_cores=NC, num_subcores=NS)
    return pl.kernel(
        _make_body(rows, n),
        out_type=(
            jax.ShapeDtypeStruct((rows, n), jnp.float32),
            jax.ShapeDtypeStruct((rows, n), jnp.int32),
        ),
        mesh=mesh,
        scratch_types=[
            pltpu.VMEM((n,), jnp.float32),
            pltpu.VMEM((n,), jnp.int32),
            pltpu.VMEM((n + L,), jnp.int32),
            pltpu.VMEM((D * L,), jnp.int32),
            pltpu.VMEM((n // L,), jnp.float32),
        ],
        compiler_params=pltpu.CompilerParams(needs_layout_passes=False),
        interpret=interpret,
    )


@jax.jit
def kernel(x):
    return _make_kernel(R, N)(x)


# speedup vs baseline: 1.8810x; 1.0023x over previous
import functools

import jax
import jax.numpy as jnp
from jax import lax
from jax.experimental import pallas as pl
from jax.experimental.pallas import tpu as pltpu
from jax.experimental.pallas import tpu_sc as plsc

R = 128
N = 32768
L = 16
NBITS = 8
D = 1 << NBITS
NPASS = 32 // NBITS
NC, NS = 2, 16
PG = 4
P_TOP = 0.9


def _digit(t, shift):
    return lax.shift_right_logical(t, jnp.int32(shift)) & jnp.int32(D - 1)


def _make_body(rows, n):
    nv = n // L
    chunk = n // L
    chunk_bits = chunk.bit_length() - 1
    rows_per_w = rows // (NC * NS)

    def _sc_body(x_hbm, vals_hbm, order_hbm, bufA, bufB, keysT, counters,
                 psums, osem):
        lane = lax.broadcasted_iota(jnp.int32, (L,), 0)
        ones = jnp.ones((L,), jnp.int32)
        wid = lax.axis_index("s") * NC + lax.axis_index("c")

        def kslot(idx):
            return idx + lax.shift_right_logical(idx, jnp.int32(chunk_bits))

        def row_body(j, _):
            r = wid * rows_per_w + j
            pltpu.sync_copy(x_hbm.at[r], bufA)

            def tr_body(i, mx):
                v = bufA[pl.ds(i * L, L)]
                u = plsc.bitcast(v, jnp.int32)
                t = jnp.where(u < 0, u, ~u & jnp.int32(0x7FFFFFFF))
                e = i * L + lane
                plsc.store_scatter(keysT, [kslot(e)], t)
                return jnp.maximum(mx, v)

            mx16 = lax.fori_loop(0, nv, tr_body,
                                 jnp.full((L,), -jnp.inf, jnp.float32),
                                 unroll=8)
            mx = jnp.max(mx16)

            def sum_body(i, acc):
                v = bufA[pl.ds(i * L, L)]
                return acc + jnp.exp(v - mx)

            acc16 = lax.fori_loop(0, nv, sum_body, jnp.zeros((L,), jnp.float32),
                                  unroll=4)
            total = jnp.sum(acc16)

            c15 = jnp.int32(15)
            c23 = jnp.int32(23)
            cD1 = jnp.int32(D - 1)
            cIM = jnp.int32(n - 1)

            for p in range(NPASS):
                last = p == NPASS - 1
                src = None if p == 0 else (bufA if p % 2 == 1 else bufB)

                if p == 1:
                    @pl.when(j > 0)
                    def _():
                        pltpu.make_async_copy(
                            bufB, order_hbm.at[r], osem).wait()

                def load_pk(i, src=src):
                    iv = src[pl.ds(i * L, L)]
                    return plsc.bitcast(iv, jnp.int32) if src is bufA else iv

                def zero_body(i, _):
                    counters[pl.ds(i * L, L)] = jnp.zeros((L,), jnp.int32)
                    return 0

                lax.fori_loop(0, D, zero_body, 0)

                if p == 0:
                    def hist_body(i, _):
                        t = plsc.load_gather(keysT, [lane * (chunk + 1) + i])
                        d = t & cD1
                        plsc.addupdate_scatter(counters, [d * L + lane], ones)
                        return 0
                else:
                    def hist_body(i, _, load_pk=load_pk):
                        pk = load_pk(i)
                        d = lax.shift_right_logical(pk, c15) & cD1
                        plsc.addupdate_scatter(counters, [d * L + lane], ones)
                        return 0

                lax.fori_loop(0, nv, hist_body, 0, unroll=4 if p == 0 else 16)

                def scan_body(g, carry):
                    vs = [counters[pl.ds((g * 4 + k) * L, L)]
                          for k in range(4)]
                    incs = [plsc.cumsum(v) for v in vs]
                    tots = [jnp.max(inc) for inc in incs]
                    for k in range(4):
                        counters[pl.ds((g * 4 + k) * L, L)] = \
                            incs[k] - vs[k] + carry
                        carry = carry + tots[k]
                    return carry

                lax.fori_loop(0, D // 4, scan_body, jnp.int32(0))

                def perm_group(g, _, p=p, last=last, load_pk=load_pk):
                    ds_, outs = [], []
                    for k in range(PG):
                        i = g * PG + k
                        if p == 0:
                            t = plsc.load_gather(
                                keysT, [lane * (chunk + 1) + i])
                            idxv = lane * chunk + i
                            d = t & cD1
                            d1 = lax.shift_right_logical(t, jnp.int32(8)) & cD1
                            d2 = lax.shift_right_logical(t, jnp.int32(16)) & cD1
                            out = (d2 << c23) | (d1 << c15) | idxv
                        else:
                            pk = load_pk(i)
                            d = lax.shift_right_logical(pk, c15) & cD1
                            idxv = pk & cIM
                            if p == 1:
                                sp = lax.shift_right_logical(pk, c23)
                                t = plsc.load_gather(keysT, [kslot(idxv)])
                                d3 = lax.shift_right_logical(t, jnp.int32(24))
                                out = (d3 << c23) | (sp << c15) | idxv
                            elif p == 2:
                                sp = lax.shift_right_logical(pk, c23)
                                out = (sp << c15) | idxv
                            else:
                                out = idxv
                        ds_.append(d)
                        outs.append(out)
                    bases = [plsc.load_gather(counters, [d * L + lane])
                             for d in ds_]
                    eqs = {}
                    for j in range(PG):
                        for k in range(j + 1, PG):
                            eqs[(j, k)] = ds_[j] == ds_[k]
                    for k in range(PG):
                        occ = jnp.zeros((L,), jnp.int32)
                        for j in range(k):
                            occ = occ + jnp.where(eqs[(j, k)], 1, 0)
                        pos = bases[k] + occ
                        is_last = jnp.full((L,), True)
                        for j in range(k + 1, PG):
                            is_last = is_last & jnp.logical_not(eqs[(k, j)])
                        plsc.store_scatter(counters, [ds_[k] * L + lane],
                                           pos + ones, mask=is_last)
                        if last:
                            s = pos
                        else:
                            s = ((pos & jnp.int32(chunk - 1)) << 4) | \
                                lax.shift_right_logical(
                                    pos, jnp.int32(chunk_bits))
                        if p % 2 == 0:
                            plsc.store_scatter(bufA, [s],
                                               plsc.bitcast(outs[k],
                                                            jnp.float32))
                        else:
                            plsc.store_scatter(bufB, [s], outs[k])
                    return 0

                lax.fori_loop(0, nv // PG, perm_group, 0)

            thresh = jnp.float32(P_TOP) * total

            MG = 8
            last_lane = jnp.full((L,), L - 1, jnp.int32)

            def mask_body(g, cum):
                vs, es, incs = [], [], []
                for k in range(MG):
                    i = g * MG + k
                    idxv = bufB[pl.ds(i * L, L)]
                    t = plsc.load_gather(keysT, [kslot(idxv)])
                    u = jnp.where(t < 0, t, ~t & jnp.int32(0x7FFFFFFF))
                    v = plsc.bitcast(u, jnp.float32)
                    e = jnp.exp(v - mx)
                    vs.append(v)
                    es.append(e)
                    incs.append(plsc.cumsum(e))
                for k in range(MG):
                    i = g * MG + k
                    excl = incs[k] - es[k] + cum
                    out = jnp.where(excl < thresh, vs[k], -jnp.inf)
                    bufA[pl.ds(i * L, L)] = out
                    cum = cum + jnp.take(incs[k], last_lane, mode="wrap")
                return cum

            lax.fori_loop(0, nv // MG, mask_body,
                          jnp.zeros((L,), jnp.float32))

            pltpu.sync_copy(bufA, vals_hbm.at[r])
            pltpu.async_copy(bufB, order_hbm.at[r], osem)
            return 0

        lax.fori_loop(0, rows_per_w, row_body, 0)
        pltpu.make_async_copy(bufB, order_hbm.at[wid * rows_per_w],
                              osem).wait()

    return _sc_body


def _make_kernel(rows, n, interpret=False):
    mesh = plsc.VectorSubcoreMesh(core_axis_name="c", subcore_axis_name="s",
                                  num_cores=NC, num_subcores=NS)
    return pl.kernel(
        _make_body(rows, n),
        out_type=(
            jax.ShapeDtypeStruct((rows, n), jnp.float32),
            jax.ShapeDtypeStruct((rows, n), jnp.int32),
        ),
        mesh=mesh,
        scratch_types=[
            pltpu.VMEM((n,), jnp.float32),
            pltpu.VMEM((n,), jnp.int32),
            pltpu.VMEM((n + L,), jnp.int32),
            pltpu.VMEM((D * L,), jnp.int32),
            pltpu.VMEM((n // L,), jnp.float32),
            pltpu.SemaphoreType.DMA,
        ],
        compiler_params=pltpu.CompilerParams(needs_layout_passes=False),
        interpret=interpret,
    )


@jax.jit
def kernel(x):
    return _make_kernel(R, N)(x)
